# baseline (speedup 1.0000x reference)
import functools

import jax
import jax.numpy as jnp
from jax import lax
from jax.experimental import pallas as pl
from jax.experimental.pallas import tpu as pltpu
from jax.experimental.pallas import tpu_sc as plsc

VOCAB = 1000000
SEQ_LEN = 200
EMBED_DIM = 64
BATCH = 4096

NC, NS, L = 2, 16, 16
NW = NC * NS
B_PER_W = BATCH // NW
CB = 4
PAIR = 2 * CB
N_PAIRS = B_PER_W // PAIR
G = 40
GPR = SEQ_LEN // G

_SCALE = 8.0


def _emb_kernel(idx_hbm, tok_hbm, pos_hbm, out_hbm,
                pos_v, idx_v, rows_a, rows_b, sg_a, sg_b, ss_a, ss_b):
    wid = lax.axis_index("s") * NC + lax.axis_index("c")
    base = wid * B_PER_W

    pltpu.sync_copy(pos_hbm, pos_v)

    bufs = (rows_a, rows_b)
    gsems = (sg_a, sg_b)
    ssems = (ss_a, ss_b)

    def gather_chunk(half, pb):
        buf, sem = bufs[half], gsems[half]
        for h in range(GPR * CB):
            pltpu.async_copy(
                tok_hbm.at[idx_v.at[half * CB + h // GPR, pl.ds((h % GPR) * G, G)]],
                buf.at[h // GPR, pl.ds((h % GPR) * G, G), :],
                sem,
            )

    def drain_gather(half):
        buf, sem = bufs[half], gsems[half]
        for h in range(GPR * CB):
            pltpu.make_async_copy(
                tok_hbm.at[idx_v.at[half * CB + h // GPR, pl.ds((h % GPR) * G, G)]],
                buf.at[h // GPR, pl.ds((h % GPR) * G, G), :],
                sem,
            ).wait()

    def drain_scatter(half, dst):
        pltpu.make_async_copy(bufs[half], dst, ssems[half]).wait()

    def compute(half):
        buf = bufs[half]

        def p_body(p, carry):
            pv = [pos_v[p, pl.ds(c * L, L)] for c in range(EMBED_DIM // L)]
            for s in range(CB):
                for c in range(EMBED_DIM // L):
                    sl = pl.ds(c * L, L)
                    buf[s, p, sl] = buf[s, p, sl] * _SCALE + pv[c]
            return carry

        lax.fori_loop(0, SEQ_LEN, p_body, 0)

    def pair_body(p, scattered):
        b0 = pl.multiple_of(base + p * PAIR, 8)
        pltpu.sync_copy(idx_hbm.at[pl.ds(b0, PAIR)], idx_v)
        for half in range(2):
            dst = out_hbm.at[pl.ds(b0 + half * CB, CB)]

            @pl.when(scattered != 0)
            def _():
                drain_scatter(half, dst)

            gather_chunk(half, p)

        for half in range(2):
            dst = out_hbm.at[pl.ds(b0 + half * CB, CB)]
            drain_gather(half)
            compute(half)
            pltpu.async_copy(bufs[half], dst, ssems[half])
        return 1

    scattered = lax.fori_loop(0, N_PAIRS, pair_body, 0)

    @pl.when(scattered != 0)
    def _():
        last = pl.multiple_of(base + (N_PAIRS - 1) * PAIR, 8)
        for half in range(2):
            drain_scatter(half, out_hbm.at[pl.ds(last + half * CB, CB)])


def _repack_body(in_ref, out_ref):
    t = jnp.transpose(in_ref[...])
    t3 = t.reshape(256, 2, 64)
    te = t3[:, 0, :]
    to = t3[:, 1, :]
    out_ref[:, 0:64] = te
    out_ref[:, 64:128] = to


def _repack(tok_t):
    nblk = (VOCAB + 511) // 512
    return pl.pallas_call(
        _repack_body,
        grid=(nblk,),
        in_specs=[pl.BlockSpec((64, 512), lambda g: (0, g))],
        out_specs=pl.BlockSpec((256, 128), lambda g: (g, 0)),
        out_shape=jax.ShapeDtypeStruct((VOCAB // 2, 128), jnp.float32),
    )(tok_t)


@jax.jit
def _run(inputs, token_table, position_table):
    mesh = plsc.VectorSubcoreMesh(
        core_axis_name="c", subcore_axis_name="s", num_cores=NC, num_subcores=NS
    )
    kern = functools.partial(
        pl.kernel,
        out_type=jax.ShapeDtypeStruct((BATCH, SEQ_LEN, EMBED_DIM), jnp.float32),
        mesh=mesh,
        scratch_types=[
            pltpu.VMEM((SEQ_LEN, EMBED_DIM), jnp.float32),
            pltpu.VMEM((PAIR, SEQ_LEN), jnp.int32),
            pltpu.VMEM((CB, SEQ_LEN, EMBED_DIM), jnp.float32),
            pltpu.VMEM((CB, SEQ_LEN, EMBED_DIM), jnp.float32),
            pltpu.SemaphoreType.DMA,
            pltpu.SemaphoreType.DMA,
            pltpu.SemaphoreType.DMA,
            pltpu.SemaphoreType.DMA,
        ],
        compiler_params=pltpu.CompilerParams(use_tc_tiling_on_sc=False),
    )(_emb_kernel)
    tok_lin = _repack(token_table.T).reshape(VOCAB, EMBED_DIM)
    return kern(inputs, tok_lin, position_table)


def kernel(inputs, token_table, position_table):
    return _run(inputs, token_table, position_table)


# speedup vs baseline: 1.0292x; 1.0292x over previous
import functools

import jax
import jax.numpy as jnp
from jax import lax
from jax.experimental import pallas as pl
from jax.experimental.pallas import tpu as pltpu
from jax.experimental.pallas import tpu_sc as plsc

VOCAB = 1000000
SEQ_LEN = 200
EMBED_DIM = 64
BATCH = 4096

NC, NS, L = 2, 16, 16
NW = NC * NS
BT = BATCH // NW
SST = 8
N_STAGES = SEQ_LEN // SST
NCH = EMBED_DIM // L

_SCALE = 8.0


def _emb_kernel(idx_hbm, tok_hbm, pos_hbm, out_hbm,
                pos_v, idx_v, rows_a, rows_b, blk_a, blk_b,
                sg_a, sg_b, so_a, so_b):
    wid = lax.axis_index("s") * NC + lax.axis_index("c")
    b0 = wid * BT

    pltpu.sync_copy(pos_hbm, pos_v)

    rbufs = (rows_a, rows_b)
    gsems = (sg_a, sg_b)
    oblks = (blk_a, blk_b)
    osems = (so_a, so_b)

    io = lax.iota(jnp.int32, L)
    abase = [(io + 16 * c) * BT for c in range(NCH)]

    def gather_pos(j):
        pltpu.async_copy(tok_hbm.at[idx_v.at[j]], rbufs[j % 2], gsems[j % 2])

    def drain_gather(j):
        pltpu.make_async_copy(
            tok_hbm.at[idx_v.at[j]], rbufs[j % 2], gsems[j % 2]
        ).wait()

    def blk_copies(s, j, blk):
        return [
            pltpu.make_async_copy(
                blk.at[pl.ds(td * 8 * BT, 8 * BT)],
                out_hbm.at[s, td, wid],
                osems[j % 2],
            )
            for td in range(EMBED_DIM // 8)
        ]

    def compute(s, j):
        buf, blk = rbufs[j % 2], oblks[j % 2]
        pv = [pos_v[s, pl.ds(c * L, L)] for c in range(NCH)]

        def b_body(b, carry):
            bb = jax.lax.broadcast(b, (L,))
            for c in range(NCH):
                v = buf[b, pl.ds(c * L, L)] * _SCALE + pv[c]
                plsc.store_scatter(blk, [abase[c] + bb], v)
            return carry

        lax.fori_loop(0, BT, b_body, 0)

    def stage_body(st, carry):
        s0 = pl.multiple_of(st * SST, 8)
        pltpu.sync_copy(idx_hbm.at[pl.ds(s0, SST), pl.ds(b0, BT)], idx_v)
        gather_pos(0)
        for j in range(SST):
            s = s0 + j
            if j < SST - 1:
                gather_pos(j + 1)
            drain_gather(j)

            @pl.when(s >= 2)
            def _():
                for cp in blk_copies(s, j, oblks[j % 2]):
                    cp.wait()

            compute(s, j)
            for cp in blk_copies(s, j, oblks[j % 2]):
                cp.start()
        return carry

    lax.fori_loop(0, N_STAGES, stage_body, 0)

    for j in range(2):
        for cp in blk_copies(SEQ_LEN - 2 + j, j, oblks[j]):
            cp.wait()


@jax.jit
def _run(inputs, token_table, position_table):
    mesh = plsc.VectorSubcoreMesh(
        core_axis_name="c", subcore_axis_name="s", num_cores=NC, num_subcores=NS
    )
    kern = functools.partial(
        pl.kernel,
        out_type=jax.ShapeDtypeStruct(
            (SEQ_LEN, EMBED_DIM // 8, NW, 8 * BT), jnp.float32
        ),
        mesh=mesh,
        scratch_types=[
            pltpu.VMEM((SEQ_LEN, EMBED_DIM), jnp.float32),
            pltpu.VMEM((SST, BT), jnp.int32),
            pltpu.VMEM((BT, EMBED_DIM), jnp.float32),
            pltpu.VMEM((BT, EMBED_DIM), jnp.float32),
            pltpu.VMEM((EMBED_DIM * BT,), jnp.float32),
            pltpu.VMEM((EMBED_DIM * BT,), jnp.float32),
            pltpu.SemaphoreType.DMA,
            pltpu.SemaphoreType.DMA,
            pltpu.SemaphoreType.DMA,
            pltpu.SemaphoreType.DMA,
        ],
        compiler_params=pltpu.CompilerParams(
            use_tc_tiling_on_sc=False, needs_layout_passes=False
        ),
    )(_emb_kernel)
    out4 = kern(inputs.T, token_table, position_table)
    out5 = out4.reshape(SEQ_LEN, EMBED_DIM // 8, NW, 8, BT)
    return out5.transpose(2, 4, 0, 1, 3).reshape(BATCH, SEQ_LEN, EMBED_DIM)


def kernel(inputs, token_table, position_table):
    return _run(inputs, token_table, position_table)


# speedup vs baseline: 1.5392x; 1.4955x over previous
import functools

import jax
import jax.numpy as jnp
from jax import lax
from jax.experimental import pallas as pl
from jax.experimental.pallas import tpu as pltpu
from jax.experimental.pallas import tpu_sc as plsc

VOCAB = 1000000
SEQ_LEN = 200
EMBED_DIM = 64
BATCH = 4096

NC, NS, L = 2, 16, 16
NW = NC * NS
B_PER_W = BATCH // NW
CB = 4
PAIR = 2 * CB
N_PAIRS = B_PER_W // PAIR
G = 40
GPR = SEQ_LEN // G

_SCALE = 8.0


def _emb_kernel(idx_hbm, tok_hbm, pos_hbm, out_hbm,
                pos_v, idx_v, rows_a, rows_b, sg_a, sg_b, ss_a, ss_b):
    wid = lax.axis_index("s") * NC + lax.axis_index("c")
    base = wid * B_PER_W

    pltpu.sync_copy(pos_hbm, pos_v)

    bufs = (rows_a, rows_b)
    gsems = (sg_a, sg_b)
    ssems = (ss_a, ss_b)

    def gather_chunk(half, pb):
        buf, sem = bufs[half], gsems[half]
        for h in range(GPR * CB):
            pltpu.async_copy(
                tok_hbm.at[idx_v.at[half * CB + h // GPR, pl.ds((h % GPR) * G, G)]],
                buf.at[h // GPR, pl.ds((h % GPR) * G, G), :],
                sem,
            )

    def drain_gather(half):
        buf, sem = bufs[half], gsems[half]
        for h in range(GPR * CB):
            pltpu.make_async_copy(
                tok_hbm.at[idx_v.at[half * CB + h // GPR, pl.ds((h % GPR) * G, G)]],
                buf.at[h // GPR, pl.ds((h % GPR) * G, G), :],
                sem,
            ).wait()

    def drain_scatter(half, dst):
        pltpu.make_async_copy(bufs[half], dst, ssems[half]).wait()

    def compute(half):
        buf = bufs[half]

        def p_body(p, carry):
            pv = [pos_v[p, pl.ds(c * L, L)] for c in range(EMBED_DIM // L)]
            for s in range(CB):
                for c in range(EMBED_DIM // L):
                    sl = pl.ds(c * L, L)
                    buf[s, p, sl] = buf[s, p, sl] * _SCALE + pv[c]
            return carry

        lax.fori_loop(0, SEQ_LEN, p_body, 0)

    def pair_body(p, scattered):
        b0 = pl.multiple_of(base + p * PAIR, 8)
        pltpu.sync_copy(idx_hbm.at[pl.ds(b0, PAIR)], idx_v)
        for half in range(2):
            dst = out_hbm.at[pl.ds(b0 + half * CB, CB)]

            @pl.when(scattered != 0)
            def _():
                drain_scatter(half, dst)

            gather_chunk(half, p)

        for half in range(2):
            dst = out_hbm.at[pl.ds(b0 + half * CB, CB)]
            drain_gather(half)
            compute(half)
            pltpu.async_copy(bufs[half], dst, ssems[half])
        return 1

    scattered = lax.fori_loop(0, N_PAIRS, pair_body, 0)

    @pl.when(scattered != 0)
    def _():
        last = pl.multiple_of(base + (N_PAIRS - 1) * PAIR, 8)
        for half in range(2):
            drain_scatter(half, out_hbm.at[pl.ds(last + half * CB, CB)])


@jax.jit
def _run(inputs, token_table, position_table):
    mesh = plsc.VectorSubcoreMesh(
        core_axis_name="c", subcore_axis_name="s", num_cores=NC, num_subcores=NS
    )
    kern = functools.partial(
        pl.kernel,
        out_type=jax.ShapeDtypeStruct((BATCH, SEQ_LEN, EMBED_DIM), jnp.float32),
        mesh=mesh,
        scratch_types=[
            pltpu.VMEM((SEQ_LEN, EMBED_DIM), jnp.float32),
            pltpu.VMEM((PAIR, SEQ_LEN), jnp.int32),
            pltpu.VMEM((CB, SEQ_LEN, EMBED_DIM), jnp.float32),
            pltpu.VMEM((CB, SEQ_LEN, EMBED_DIM), jnp.float32),
            pltpu.SemaphoreType.DMA,
            pltpu.SemaphoreType.DMA,
            pltpu.SemaphoreType.DMA,
            pltpu.SemaphoreType.DMA,
        ],
        compiler_params=pltpu.CompilerParams(use_tc_tiling_on_sc=False),
    )(_emb_kernel)
    return kern(inputs, token_table, position_table)


def kernel(inputs, token_table, position_table):
    return _run(inputs, token_table, position_table)
